# baseline (speedup 1.0000x reference)
import functools

import jax
import jax.numpy as jnp
from jax import lax
from jax.experimental import pallas as pl
from jax.experimental.pallas import tpu as pltpu
from jax.experimental.pallas import tpu_sc as plsc

_CHUNK = 128


@functools.cache
def _build(V, D, B):
    info = plsc.get_sparse_core_info()
    NC, NS = info.num_cores, info.num_subcores
    NW = NC * NS
    b_per_w = B // NW
    n_ch = b_per_w // _CHUNK
    mesh = plsc.VectorSubcoreMesh(core_axis_name="c", subcore_axis_name="s")

    @functools.partial(
        pl.kernel,
        mesh=mesh,
        out_type=jax.ShapeDtypeStruct((D, B), jnp.float32),
        compiler_params=pltpu.CompilerParams(use_tc_tiling_on_sc=False),
        scratch_types=[
            pltpu.VMEM((n_ch, _CHUNK), jnp.int32),
            pltpu.VMEM((D, b_per_w), jnp.float32),
            pltpu.SemaphoreType.DMA,
            pltpu.SemaphoreType.DMA,
        ],
    )
    def k(ids_hbm, tableT_hbm, outT_hbm, idx_v, rowsT_v, sem, sem_i):
        wid = lax.axis_index("s") * NC + lax.axis_index("c")
        base = wid * b_per_w

        pltpu.async_copy(ids_hbm.at[wid], idx_v, sem_i).wait()

        for c in range(D):
            row_ref = tableT_hbm.at[c]
            for j in range(n_ch):
                pltpu.async_copy(
                    row_ref.at[idx_v.at[j]],
                    rowsT_v.at[c, pl.ds(j * _CHUNK, _CHUNK)],
                    sem,
                )

        pltpu.make_async_copy(
            outT_hbm.at[:, pl.ds(base, b_per_w)], rowsT_v, sem).wait()

        pltpu.sync_copy(rowsT_v, outT_hbm.at[:, pl.ds(base, b_per_w)])

    return k


def kernel(ids, table):
    B, = ids.shape
    V, D = table.shape
    info = plsc.get_sparse_core_info()
    NW = info.num_cores * info.num_subcores
    b_per_w = B // NW
    ids3 = ids.astype(jnp.int32).reshape(NW, b_per_w // _CHUNK, _CHUNK)
    outT = _build(V, D, B)(ids3, table.T)
    return outT.T


# speedup vs baseline: 13.8818x; 13.8818x over previous
import functools

import jax
import jax.numpy as jnp
from jax import lax
from jax.experimental import pallas as pl
from jax.experimental.pallas import tpu as pltpu
from jax.experimental.pallas import tpu_sc as plsc


@functools.cache
def _build(V, D, B):
    info = plsc.get_sparse_core_info()
    NC, NS = info.num_cores, info.num_subcores
    NW = NC * NS
    b_per_w = B // NW
    n_grp = b_per_w // 16
    mesh = plsc.VectorSubcoreMesh(core_axis_name="c", subcore_axis_name="s")

    @functools.partial(
        pl.kernel,
        mesh=mesh,
        out_type=jax.ShapeDtypeStruct((B, D), jnp.float32),
        scratch_types=[
            pltpu.VMEM((b_per_w,), jnp.int32),
            pltpu.VMEM((b_per_w, D), jnp.float32),
            pltpu.SemaphoreType.DMA,
            pltpu.SemaphoreType.DMA,
        ],
    )
    def k(ids_hbm, table_hbm, out_hbm, idx_v, rows_v, sem, sem_i):
        wid = lax.axis_index("s") * NC + lax.axis_index("c")
        base = wid * b_per_w

        pltpu.async_copy(ids_hbm.at[wid], idx_v, sem_i).wait()

        def fire_body(g, _):
            idvec = idx_v[pl.ds(g * 16, 16)]
            for i in range(16):
                r = idvec[i]
                pltpu.async_copy(
                    table_hbm.at[pl.ds(r, 1)],
                    rows_v.at[pl.ds(g * 16 + i, 1)],
                    sem,
                )
            return 0

        lax.fori_loop(0, n_grp, fire_body, 0)

        pltpu.make_async_copy(
            out_hbm.at[pl.ds(base, b_per_w)], rows_v, sem).wait()

        pltpu.sync_copy(rows_v, out_hbm.at[pl.ds(base, b_per_w)])

    return k


def kernel(ids, table):
    B, = ids.shape
    V, D = table.shape
    info = plsc.get_sparse_core_info()
    NW = info.num_cores * info.num_subcores
    ids2 = ids.astype(jnp.int32).reshape(NW, B // NW)
    return _build(V, D, B)(ids2, table)
